# baseline (speedup 1.0000x reference)
import functools

import jax
import jax.numpy as jnp
from jax import lax
from jax.experimental import pallas as pl
from jax.experimental.pallas import tpu as pltpu
from jax.experimental.pallas import tpu_sc as plsc

N_NODES = 10000
N_PAD = 10240
E_EDGES = 320000
H_DIM = 128
XW = 16
NC, NS = 2, 16
NW = NC * NS
CHUNK = 128
NCHUNK = E_EDGES // CHUNK
STEPS = (NCHUNK + NW - 1) // NW
ROWS_PER_SUB = N_PAD // NS
EB = 512
E_ROWS = E_EDGES // EB
CPR = EB // CHUNK

_f32 = jnp.float32
_mesh = plsc.VectorSubcoreMesh(core_axis_name="c", subcore_axis_name="s")


@functools.partial(
    pl.kernel,
    out_type=(
        jax.ShapeDtypeStruct((E_EDGES, H_DIM), _f32),
        jax.ShapeDtypeStruct((E_EDGES, H_DIM), _f32),
        jax.ShapeDtypeStruct((E_ROWS, 1, EB), _f32),
        jax.ShapeDtypeStruct((E_ROWS, 1, EB), _f32),
        jax.ShapeDtypeStruct((E_ROWS, 1, EB), _f32),
        jax.ShapeDtypeStruct((E_ROWS, 1, EB), _f32),
    ),
    mesh=_mesh,
    scratch_types=[
        pltpu.VMEM((4 * N_NODES,), _f32),
        pltpu.VMEM((CHUNK,), jnp.int32),
        pltpu.VMEM((CHUNK,), jnp.int32),
        pltpu.VMEM((CHUNK, H_DIM), _f32),
        pltpu.VMEM((CHUNK, H_DIM), _f32),
        pltpu.VMEM((1, CHUNK), _f32),
        pltpu.VMEM((1, CHUNK), _f32),
        pltpu.VMEM((1, CHUNK), _f32),
        pltpu.VMEM((1, CHUNK), _f32),
        pltpu.SemaphoreType.DMA,
    ],
    compiler_params=pltpu.CompilerParams(needs_layout_passes=False),
)
def _sc_gather(p_hbm, xflat_hbm, row_hbm, col_hbm, q_hbm,
               hi_out, hj_out, dx_out, dy_out, dz_out, d2_out,
               xtab, rowv, colv, hbi, hbj, dxb, dyb, dzb, d2b, sem):
  wid = lax.axis_index("s") * NC + lax.axis_index("c")
  pltpu.sync_copy(xflat_hbm, xtab)

  def body(t, _):
    ci = wid + t * NW

    @pl.when(ci < NCHUNK)
    def _():
      base = ci * CHUNK
      pltpu.sync_copy(row_hbm.at[pl.ds(base, CHUNK)], rowv)
      pltpu.sync_copy(col_hbm.at[pl.ds(base, CHUNK)], colv)
      c1 = pltpu.async_copy(p_hbm.at[colv], hbi, sem)
      c2 = pltpu.async_copy(q_hbm.at[rowv], hbj, sem)
      for g in range(CHUNK // 16):
        r16 = rowv[pl.ds(g * 16, 16)] * 4
        c16 = colv[pl.ds(g * 16, 16)] * 4
        dx = plsc.load_gather(xtab, [c16]) - plsc.load_gather(xtab, [r16])
        dy = (plsc.load_gather(xtab, [c16 + 1])
              - plsc.load_gather(xtab, [r16 + 1]))
        dz = (plsc.load_gather(xtab, [c16 + 2])
              - plsc.load_gather(xtab, [r16 + 2]))
        dxb[0, pl.ds(g * 16, 16)] = dx
        dyb[0, pl.ds(g * 16, 16)] = dy
        dzb[0, pl.ds(g * 16, 16)] = dz
        d2b[0, pl.ds(g * 16, 16)] = dx * dx + dy * dy + dz * dz
      c1.wait()
      c2.wait()
      pltpu.sync_copy(hbi, hi_out.at[pl.ds(base, CHUNK)])
      pltpu.sync_copy(hbj, hj_out.at[pl.ds(base, CHUNK)])
      er = ci // CPR
      ec = (ci % CPR) * CHUNK
      pltpu.sync_copy(dxb, dx_out.at[er, pl.ds(0, 1), pl.ds(ec, CHUNK)])
      pltpu.sync_copy(dyb, dy_out.at[er, pl.ds(0, 1), pl.ds(ec, CHUNK)])
      pltpu.sync_copy(dzb, dz_out.at[er, pl.ds(0, 1), pl.ds(ec, CHUNK)])
      pltpu.sync_copy(d2b, d2_out.at[er, pl.ds(0, 1), pl.ds(ec, CHUNK)])
    return 0

  lax.fori_loop(0, STEPS, body, 0)


@functools.partial(
    pl.kernel,
    out_type=(
        jax.ShapeDtypeStruct((NC, N_PAD, H_DIM), _f32),
        jax.ShapeDtypeStruct((NW, 3 * N_PAD), _f32),
    ),
    mesh=_mesh,
    scratch_types=[
        pltpu.VMEM((CHUNK,), jnp.int32),
        pltpu.VMEM((CHUNK, H_DIM), _f32),
        pltpu.VMEM((CHUNK,), _f32),
        pltpu.VMEM((CHUNK,), _f32),
        pltpu.VMEM((CHUNK,), _f32),
        pltpu.VMEM((CHUNK,), _f32),
        pltpu.VMEM((3 * N_PAD,), _f32),
        pltpu.VMEM_SHARED((N_PAD, H_DIM), _f32),
        pltpu.SemaphoreType.DMA,
    ],
    compiler_params=pltpu.CompilerParams(needs_layout_passes=False),
)
def _sc_scatter(m_hbm, cw_hbm, dx_hbm, dy_hbm, dz_hbm, col_hbm, z_hbm, zc_hbm,
                magg_out, cagg_out,
                colv, mbuf, cwb, dxb, dyb, dzb, cacc, macc, sem):
  cid = lax.axis_index("c")
  sid = lax.axis_index("s")
  wid = sid * NC + cid
  rbase = sid * ROWS_PER_SUB

  pltpu.sync_copy(z_hbm.at[pl.ds(rbase, ROWS_PER_SUB)],
                  macc.at[pl.ds(rbase, ROWS_PER_SUB)])
  pltpu.sync_copy(zc_hbm, cacc)
  plsc.subcore_barrier()

  def body(t, _):
    ci = wid + t * NW

    @pl.when(ci < NCHUNK)
    def _():
      base = ci * CHUNK
      pltpu.sync_copy(col_hbm.at[pl.ds(base, CHUNK)], colv)
      pltpu.sync_copy(m_hbm.at[pl.ds(base, CHUNK)], mbuf)
      er = ci // CPR
      ec = (ci % CPR) * CHUNK
      pltpu.sync_copy(cw_hbm.at[er, 0, pl.ds(ec, CHUNK)], cwb)
      pltpu.sync_copy(dx_hbm.at[er, 0, pl.ds(ec, CHUNK)], dxb)
      pltpu.sync_copy(dy_hbm.at[er, 0, pl.ds(ec, CHUNK)], dyb)
      pltpu.sync_copy(dz_hbm.at[er, 0, pl.ds(ec, CHUNK)], dzb)
      pltpu.sync_copy(mbuf, macc.at[colv], add=True)
      for g in range(CHUNK // 16):
        sl = pl.ds(g * 16, 16)
        c3 = colv[sl] * 3
        cw = cwb[sl]
        plsc.addupdate_scatter(cacc, [c3], dxb[sl] * cw)
        plsc.addupdate_scatter(cacc, [c3 + 1], dyb[sl] * cw)
        plsc.addupdate_scatter(cacc, [c3 + 2], dzb[sl] * cw)
    return 0

  lax.fori_loop(0, STEPS, body, 0)
  plsc.subcore_barrier()

  pltpu.sync_copy(macc.at[pl.ds(rbase, ROWS_PER_SUB)],
                  magg_out.at[cid, pl.ds(rbase, ROWS_PER_SUB)])
  pltpu.sync_copy(cacc, cagg_out.at[wid])


def _pack_bf16(v):
  r = v.astype(jnp.bfloat16).astype(_f32)
  bits = lax.bitcast_convert_type(r, jnp.int32)
  lo = lax.shift_right_logical(bits[:, :64], 16)
  return lo | bits[:, 64:]


def _unpack_bf16(w):
  lo = lax.bitcast_convert_type(w << 16, _f32)
  hi = lax.bitcast_convert_type(w & jnp.int32(-65536), _f32)
  return jnp.concatenate([lo, hi], axis=-1)


def _prep_body(h, A, B, p_out, q_out):
  p_out[...] = jnp.dot(h[...], A[...], preferred_element_type=_f32)
  q_out[...] = jnp.dot(h[...], B[...], preferred_element_type=_f32)


def _edge_body(hi, hj, d2, ea, mask,
               wd, C, be1, We2, be2, wg, bg, wc, bc,
               m_out, cw_out):
  d2c = d2[...].reshape(EB, 1)
  dist = jnp.sqrt(d2c + 1e-8)
  t = (hi[...] + hj[...]
       + dist * wd[...]
       + jnp.dot(ea[...], C[...], preferred_element_type=_f32)
       + be1[...])
  m1 = t * jax.nn.sigmoid(t)
  u = jnp.dot(m1, We2[...], preferred_element_type=_f32) + be2[...]
  m2 = u * jax.nn.sigmoid(u)
  gate = jax.nn.sigmoid(
      jnp.sum(m2 * wg[...], axis=-1, keepdims=True) + bg[...])
  mg = m2 * (gate * mask[...])
  cw = jnp.sum(mg * wc[...], axis=-1, keepdims=True) + bc[...]
  m_out[...] = mg
  cw_out[...] = cw.reshape(1, 1, EB)


def _node_body(h, p0, p1, cp, xp,
               Wn1a, Wn1b, bn1, Wn2, bn2,
               h_out, x_out):
  ma = p0[0] + p1[0]
  t = (jnp.dot(h[...], Wn1a[...], preferred_element_type=_f32)
       + jnp.dot(ma, Wn1b[...], preferred_element_type=_f32)
       + bn1[...])
  s = t * jax.nn.sigmoid(t)
  dh = jnp.dot(s, Wn2[...], preferred_element_type=_f32) + bn2[...]
  h_out[...] = h[...] + dh
  x_out[...] = xp[...] + jnp.sum(cp[...], axis=0)


def _full_spec(shape):
  return pl.BlockSpec(shape, lambda i: tuple(0 for _ in shape))


def kernel(h, x, edge_index, edge_mask, edge_attr,
           We1, be1, We2, be2, Wg, bg, Wn1, bn1, Wn2, bn2, Wc, bc):
  row = edge_index[0]
  col = edge_index[1]
  x_flat = jnp.pad(x, ((0, 0), (0, 1))).reshape(-1)

  A = We1[:H_DIM]
  B = We1[H_DIM:2 * H_DIM]
  wd = We1[2 * H_DIM:2 * H_DIM + 1]
  C = We1[2 * H_DIM + 1:]

  h_pad = jnp.pad(h, ((0, N_PAD - N_NODES), (0, 0)))
  NB = 512
  ngrid = N_PAD // NB
  P, Q = pl.pallas_call(
      _prep_body,
      grid=(ngrid,),
      in_specs=[
          pl.BlockSpec((NB, H_DIM), lambda i: (i, 0)),
          _full_spec((H_DIM, H_DIM)),
          _full_spec((H_DIM, H_DIM)),
      ],
      out_specs=[
          pl.BlockSpec((NB, H_DIM), lambda i: (i, 0)),
          pl.BlockSpec((NB, H_DIM), lambda i: (i, 0)),
      ],
      out_shape=[
          jax.ShapeDtypeStruct((N_PAD, H_DIM), _f32),
          jax.ShapeDtypeStruct((N_PAD, H_DIM), _f32),
      ],
      compiler_params=pltpu.CompilerParams(
          dimension_semantics=("arbitrary",)),
  )(h_pad, A, B)

  hi, hj, dx, dy, dz, d2 = _sc_gather(P, x_flat, row, col, Q)

  egrid = E_EDGES // EB
  m_ij, cw = pl.pallas_call(
      _edge_body,
      grid=(egrid,),
      in_specs=[
          pl.BlockSpec((EB, H_DIM), lambda i: (i, 0)),
          pl.BlockSpec((EB, H_DIM), lambda i: (i, 0)),
          pl.BlockSpec((1, 1, EB), lambda i: (i, 0, 0)),
          pl.BlockSpec((EB, 16), lambda i: (i, 0)),
          pl.BlockSpec((EB, 1), lambda i: (i, 0)),
          _full_spec((1, H_DIM)),
          _full_spec((16, H_DIM)),
          _full_spec((1, H_DIM)),
          _full_spec((H_DIM, H_DIM)),
          _full_spec((1, H_DIM)),
          _full_spec((1, H_DIM)),
          _full_spec((1, 1)),
          _full_spec((1, H_DIM)),
          _full_spec((1, 1)),
      ],
      out_specs=[
          pl.BlockSpec((EB, H_DIM), lambda i: (i, 0)),
          pl.BlockSpec((1, 1, EB), lambda i: (i, 0, 0)),
      ],
      out_shape=[
          jax.ShapeDtypeStruct((E_EDGES, H_DIM), _f32),
          jax.ShapeDtypeStruct((E_ROWS, 1, EB), _f32),
      ],
      compiler_params=pltpu.CompilerParams(
          dimension_semantics=("arbitrary",)),
  )(hi, hj, d2, edge_attr, edge_mask,
    wd, C, be1.reshape(1, -1), We2, be2.reshape(1, -1),
    Wg.reshape(1, -1), bg.reshape(1, 1), Wc.reshape(1, -1), bc.reshape(1, 1))

  zeros = jnp.zeros((N_PAD, H_DIM), _f32)
  zeros_c = jnp.zeros((3 * N_PAD,), _f32)
  magg, cagg = _sc_scatter(m_ij, cw, dx, dy, dz, col, zeros, zeros_c)
  cagg = cagg.reshape(NW, N_PAD, 3)

  xp_pad = jnp.pad(x, ((0, N_PAD - N_NODES), (0, 0)))

  h_out, x_out = pl.pallas_call(
      _node_body,
      grid=(ngrid,),
      in_specs=[
          pl.BlockSpec((NB, H_DIM), lambda i: (i, 0)),
          pl.BlockSpec((1, NB, H_DIM), lambda i: (0, i, 0)),
          pl.BlockSpec((1, NB, H_DIM), lambda i: (1, i, 0)),
          pl.BlockSpec((NW, NB, 3), lambda i: (0, i, 0)),
          pl.BlockSpec((NB, 3), lambda i: (i, 0)),
          _full_spec((H_DIM, H_DIM)),
          _full_spec((H_DIM, H_DIM)),
          _full_spec((1, H_DIM)),
          _full_spec((H_DIM, H_DIM)),
          _full_spec((1, H_DIM)),
      ],
      out_specs=[
          pl.BlockSpec((NB, H_DIM), lambda i: (i, 0)),
          pl.BlockSpec((NB, 3), lambda i: (i, 0)),
      ],
      out_shape=[
          jax.ShapeDtypeStruct((N_PAD, H_DIM), _f32),
          jax.ShapeDtypeStruct((N_PAD, 3), _f32),
      ],
      compiler_params=pltpu.CompilerParams(
          dimension_semantics=("arbitrary",)),
  )(h_pad, magg, magg, cagg, xp_pad,
    Wn1[:H_DIM], Wn1[H_DIM:], bn1.reshape(1, -1), Wn2, bn2.reshape(1, -1))

  return (h_out[:N_NODES], x_out[:N_NODES])


# speedup vs baseline: 1.0555x; 1.0555x over previous
import functools

import jax
import jax.numpy as jnp
from jax import lax
from jax.experimental import pallas as pl
from jax.experimental.pallas import tpu as pltpu
from jax.experimental.pallas import tpu_sc as plsc

N_NODES = 10000
N_PAD = 10240
E_EDGES = 320000
H_DIM = 128
NC, NS = 2, 16
NW = NC * NS
CHUNK = 128
ROWS_PER_SUB = N_PAD // NS
EB = 512
CPR = EB // CHUNK
E_A = 312 * EB
E_B = E_EDGES - E_A

_f32 = jnp.float32
_i32 = jnp.int32
_mesh = plsc.VectorSubcoreMesh(core_axis_name="c", subcore_axis_name="s")


def _make_gather(ne):
  nchunk = ne // CHUNK
  steps = (nchunk + NW - 1) // NW
  erows = ne // EB

  @functools.partial(
      pl.kernel,
      out_type=(
          jax.ShapeDtypeStruct((ne, H_DIM), _f32),
          jax.ShapeDtypeStruct((ne, H_DIM), _f32),
          jax.ShapeDtypeStruct((erows, 1, EB), _f32),
          jax.ShapeDtypeStruct((erows, 1, EB), _f32),
          jax.ShapeDtypeStruct((erows, 1, EB), _f32),
          jax.ShapeDtypeStruct((erows, 1, EB), _f32),
      ),
      mesh=_mesh,
      scratch_types=[
          pltpu.VMEM((4 * N_NODES,), _f32),
          pltpu.VMEM((CHUNK,), _i32),
          pltpu.VMEM((CHUNK,), _i32),
          pltpu.VMEM((CHUNK, H_DIM), _f32),
          pltpu.VMEM((CHUNK, H_DIM), _f32),
          pltpu.VMEM((1, CHUNK), _f32),
          pltpu.VMEM((1, CHUNK), _f32),
          pltpu.VMEM((1, CHUNK), _f32),
          pltpu.VMEM((1, CHUNK), _f32),
          pltpu.SemaphoreType.DMA,
      ],
      compiler_params=pltpu.CompilerParams(needs_layout_passes=False),
  )
  def gather_k(p_hbm, xflat_hbm, row_hbm, col_hbm, q_hbm,
               hi_out, hj_out, dx_out, dy_out, dz_out, d2_out,
               xtab, rowv, colv, hbi, hbj, dxb, dyb, dzb, d2b, sem):
    wid = lax.axis_index("s") * NC + lax.axis_index("c")
    pltpu.sync_copy(xflat_hbm, xtab)

    def body(t, _):
      ci = wid + t * NW

      @pl.when(ci < nchunk)
      def _():
        base = ci * CHUNK
        pltpu.sync_copy(row_hbm.at[pl.ds(base, CHUNK)], rowv)
        pltpu.sync_copy(col_hbm.at[pl.ds(base, CHUNK)], colv)
        c1 = pltpu.async_copy(p_hbm.at[colv], hbi, sem)
        c2 = pltpu.async_copy(q_hbm.at[rowv], hbj, sem)
        for g in range(CHUNK // 16):
          r16 = rowv[pl.ds(g * 16, 16)] * 4
          c16 = colv[pl.ds(g * 16, 16)] * 4
          dx = plsc.load_gather(xtab, [c16]) - plsc.load_gather(xtab, [r16])
          dy = (plsc.load_gather(xtab, [c16 + 1])
                - plsc.load_gather(xtab, [r16 + 1]))
          dz = (plsc.load_gather(xtab, [c16 + 2])
                - plsc.load_gather(xtab, [r16 + 2]))
          dxb[0, pl.ds(g * 16, 16)] = dx
          dyb[0, pl.ds(g * 16, 16)] = dy
          dzb[0, pl.ds(g * 16, 16)] = dz
          d2b[0, pl.ds(g * 16, 16)] = dx * dx + dy * dy + dz * dz
        c1.wait()
        c2.wait()
        pltpu.sync_copy(hbi, hi_out.at[pl.ds(base, CHUNK)])
        pltpu.sync_copy(hbj, hj_out.at[pl.ds(base, CHUNK)])
        er = ci // CPR
        ec = (ci % CPR) * CHUNK
        pltpu.sync_copy(dxb, dx_out.at[er, pl.ds(0, 1), pl.ds(ec, CHUNK)])
        pltpu.sync_copy(dyb, dy_out.at[er, pl.ds(0, 1), pl.ds(ec, CHUNK)])
        pltpu.sync_copy(dzb, dz_out.at[er, pl.ds(0, 1), pl.ds(ec, CHUNK)])
        pltpu.sync_copy(d2b, d2_out.at[er, pl.ds(0, 1), pl.ds(ec, CHUNK)])
      return 0

    lax.fori_loop(0, steps, body, 0)

  return gather_k


def _make_scatter(ne):
  nchunk = ne // CHUNK
  steps = (nchunk + NW - 1) // NW

  @functools.partial(
      pl.kernel,
      out_type=(
          jax.ShapeDtypeStruct((NC, N_PAD, H_DIM), _f32),
          jax.ShapeDtypeStruct((NW, 3 * N_PAD), _f32),
      ),
      mesh=_mesh,
      scratch_types=[
          pltpu.VMEM((CHUNK,), _i32),
          pltpu.VMEM((CHUNK, H_DIM), _f32),
          pltpu.VMEM((CHUNK,), _f32),
          pltpu.VMEM((CHUNK,), _f32),
          pltpu.VMEM((CHUNK,), _f32),
          pltpu.VMEM((CHUNK,), _f32),
          pltpu.VMEM((3 * N_PAD,), _f32),
          pltpu.VMEM_SHARED((N_PAD, H_DIM), _f32),
          pltpu.SemaphoreType.DMA,
      ],
      compiler_params=pltpu.CompilerParams(needs_layout_passes=False),
  )
  def scatter_k(m_hbm, cw_hbm, dx_hbm, dy_hbm, dz_hbm, col_hbm, z_hbm, zc_hbm,
                magg_out, cagg_out,
                colv, mbuf, cwb, dxb, dyb, dzb, cacc, macc, sem):
    cid = lax.axis_index("c")
    sid = lax.axis_index("s")
    wid = sid * NC + cid
    rbase = sid * ROWS_PER_SUB

    pltpu.sync_copy(z_hbm.at[pl.ds(rbase, ROWS_PER_SUB)],
                    macc.at[pl.ds(rbase, ROWS_PER_SUB)])
    pltpu.sync_copy(zc_hbm, cacc)
    plsc.subcore_barrier()

    def body(t, _):
      ci = wid + t * NW

      @pl.when(ci < nchunk)
      def _():
        base = ci * CHUNK
        pltpu.sync_copy(col_hbm.at[pl.ds(base, CHUNK)], colv)
        pltpu.sync_copy(m_hbm.at[pl.ds(base, CHUNK)], mbuf)
        er = ci // CPR
        ec = (ci % CPR) * CHUNK
        pltpu.sync_copy(cw_hbm.at[er, 0, pl.ds(ec, CHUNK)], cwb)
        pltpu.sync_copy(dx_hbm.at[er, 0, pl.ds(ec, CHUNK)], dxb)
        pltpu.sync_copy(dy_hbm.at[er, 0, pl.ds(ec, CHUNK)], dyb)
        pltpu.sync_copy(dz_hbm.at[er, 0, pl.ds(ec, CHUNK)], dzb)
        pltpu.sync_copy(mbuf, macc.at[colv], add=True)
        for g in range(CHUNK // 16):
          sl = pl.ds(g * 16, 16)
          c3 = colv[sl] * 3
          cw = cwb[sl]
          plsc.addupdate_scatter(cacc, [c3], dxb[sl] * cw)
          plsc.addupdate_scatter(cacc, [c3 + 1], dyb[sl] * cw)
          plsc.addupdate_scatter(cacc, [c3 + 2], dzb[sl] * cw)
      return 0

    lax.fori_loop(0, steps, body, 0)
    plsc.subcore_barrier()

    pltpu.sync_copy(macc.at[pl.ds(rbase, ROWS_PER_SUB)],
                    magg_out.at[cid, pl.ds(rbase, ROWS_PER_SUB)])
    pltpu.sync_copy(cacc, cagg_out.at[wid])

  return scatter_k


_GATHER = {E_A: _make_gather(E_A), E_B: _make_gather(E_B)}
_SCATTER = {E_A: _make_scatter(E_A), E_B: _make_scatter(E_B)}


def _prep_body(h, A, B, p_out, q_out):
  p_out[...] = jnp.dot(h[...], A[...], preferred_element_type=_f32)
  q_out[...] = jnp.dot(h[...], B[...], preferred_element_type=_f32)


def _edge_body(hi, hj, d2, ea, mask,
               wd, C, be1, We2, be2, wg, bg, wc, bc,
               m_out, cw_out):
  d2c = d2[...].reshape(EB, 1)
  dist = jnp.sqrt(d2c + 1e-8)
  t = (hi[...] + hj[...]
       + dist * wd[...]
       + jnp.dot(ea[...], C[...], preferred_element_type=_f32)
       + be1[...])
  m1 = t * jax.nn.sigmoid(t)
  u = jnp.dot(m1, We2[...], preferred_element_type=_f32) + be2[...]
  m2 = u * jax.nn.sigmoid(u)
  gate = jax.nn.sigmoid(
      jnp.sum(m2 * wg[...], axis=-1, keepdims=True) + bg[...])
  mg = m2 * (gate * mask[...])
  cw = jnp.sum(mg * wc[...], axis=-1, keepdims=True) + bc[...]
  m_out[...] = mg
  cw_out[...] = cw.reshape(1, 1, EB)


def _full_spec(shape):
  return pl.BlockSpec(shape, lambda i: tuple(0 for _ in shape))


def _edge_call(hi, hj, d2, ea, mask, weights):
  ne = hi.shape[0]
  erows = ne // EB
  return pl.pallas_call(
      _edge_body,
      grid=(erows,),
      in_specs=[
          pl.BlockSpec((EB, H_DIM), lambda i: (i, 0)),
          pl.BlockSpec((EB, H_DIM), lambda i: (i, 0)),
          pl.BlockSpec((1, 1, EB), lambda i: (i, 0, 0)),
          pl.BlockSpec((EB, 16), lambda i: (i, 0)),
          pl.BlockSpec((EB, 1), lambda i: (i, 0)),
          _full_spec((1, H_DIM)),
          _full_spec((16, H_DIM)),
          _full_spec((1, H_DIM)),
          _full_spec((H_DIM, H_DIM)),
          _full_spec((1, H_DIM)),
          _full_spec((1, H_DIM)),
          _full_spec((1, 1)),
          _full_spec((1, H_DIM)),
          _full_spec((1, 1)),
      ],
      out_specs=[
          pl.BlockSpec((EB, H_DIM), lambda i: (i, 0)),
          pl.BlockSpec((1, 1, EB), lambda i: (i, 0, 0)),
      ],
      out_shape=[
          jax.ShapeDtypeStruct((ne, H_DIM), _f32),
          jax.ShapeDtypeStruct((erows, 1, EB), _f32),
      ],
      compiler_params=pltpu.CompilerParams(
          dimension_semantics=("arbitrary",)),
  )(hi, hj, d2, ea, mask, *weights)


def _node_body(h, pa0, pa1, pb0, pb1, cpa, cpb, xp,
               Wn1a, Wn1b, bn1, Wn2, bn2,
               h_out, x_out):
  ma = (pa0[0] + pa1[0]) + (pb0[0] + pb1[0])
  t = (jnp.dot(h[...], Wn1a[...], preferred_element_type=_f32)
       + jnp.dot(ma, Wn1b[...], preferred_element_type=_f32)
       + bn1[...])
  s = t * jax.nn.sigmoid(t)
  dh = jnp.dot(s, Wn2[...], preferred_element_type=_f32) + bn2[...]
  h_out[...] = h[...] + dh
  x_out[...] = xp[...] + jnp.sum(cpa[...], axis=0) + jnp.sum(cpb[...], axis=0)


def kernel(h, x, edge_index, edge_mask, edge_attr,
           We1, be1, We2, be2, Wg, bg, Wn1, bn1, Wn2, bn2, Wc, bc):
  row = edge_index[0]
  col = edge_index[1]
  x_flat = jnp.pad(x, ((0, 0), (0, 1))).reshape(-1)

  A = We1[:H_DIM]
  B = We1[H_DIM:2 * H_DIM]
  wd = We1[2 * H_DIM:2 * H_DIM + 1]
  C = We1[2 * H_DIM + 1:]
  ew = (wd, C, be1.reshape(1, -1), We2, be2.reshape(1, -1),
        Wg.reshape(1, -1), bg.reshape(1, 1), Wc.reshape(1, -1),
        bc.reshape(1, 1))

  h_pad = jnp.pad(h, ((0, N_PAD - N_NODES), (0, 0)))
  NB = 512
  ngrid = N_PAD // NB
  P, Q = pl.pallas_call(
      _prep_body,
      grid=(ngrid,),
      in_specs=[
          pl.BlockSpec((NB, H_DIM), lambda i: (i, 0)),
          _full_spec((H_DIM, H_DIM)),
          _full_spec((H_DIM, H_DIM)),
      ],
      out_specs=[
          pl.BlockSpec((NB, H_DIM), lambda i: (i, 0)),
          pl.BlockSpec((NB, H_DIM), lambda i: (i, 0)),
      ],
      out_shape=[
          jax.ShapeDtypeStruct((N_PAD, H_DIM), _f32),
          jax.ShapeDtypeStruct((N_PAD, H_DIM), _f32),
      ],
      compiler_params=pltpu.CompilerParams(
          dimension_semantics=("arbitrary",)),
  )(h_pad, A, B)

  zeros = jnp.zeros((N_PAD, H_DIM), _f32)
  zeros_c = jnp.zeros((3 * N_PAD,), _f32)

  rows = (row[:E_A], row[E_A:])
  cols = (col[:E_A], col[E_A:])
  eas = (edge_attr[:E_A], edge_attr[E_A:])
  masks = (edge_mask[:E_A], edge_mask[E_A:])
  sizes = (E_A, E_B)

  gat = [None, None]
  emlp = [None, None]
  scat = [None, None]
  gat[0] = _GATHER[E_A](P, x_flat, rows[0], cols[0], Q)
  gat[1] = _GATHER[E_B](P, x_flat, rows[1], cols[1], Q)
  for half in range(2):
    hi, hj, dx, dy, dz, d2 = gat[half]
    emlp[half] = _edge_call(hi, hj, d2, eas[half], masks[half], ew)
    m_ij, cw = emlp[half]
    scat[half] = _SCATTER[sizes[half]](m_ij, cw, dx, dy, dz, cols[half],
                                       zeros, zeros_c)

  magg_a, cagg_a = scat[0]
  magg_b, cagg_b = scat[1]
  cagg_a = cagg_a.reshape(NW, N_PAD, 3)
  cagg_b = cagg_b.reshape(NW, N_PAD, 3)

  xp_pad = jnp.pad(x, ((0, N_PAD - N_NODES), (0, 0)))

  h_out, x_out = pl.pallas_call(
      _node_body,
      grid=(ngrid,),
      in_specs=[
          pl.BlockSpec((NB, H_DIM), lambda i: (i, 0)),
          pl.BlockSpec((1, NB, H_DIM), lambda i: (0, i, 0)),
          pl.BlockSpec((1, NB, H_DIM), lambda i: (1, i, 0)),
          pl.BlockSpec((1, NB, H_DIM), lambda i: (0, i, 0)),
          pl.BlockSpec((1, NB, H_DIM), lambda i: (1, i, 0)),
          pl.BlockSpec((NW, NB, 3), lambda i: (0, i, 0)),
          pl.BlockSpec((NW, NB, 3), lambda i: (0, i, 0)),
          pl.BlockSpec((NB, 3), lambda i: (i, 0)),
          _full_spec((H_DIM, H_DIM)),
          _full_spec((H_DIM, H_DIM)),
          _full_spec((1, H_DIM)),
          _full_spec((H_DIM, H_DIM)),
          _full_spec((1, H_DIM)),
      ],
      out_specs=[
          pl.BlockSpec((NB, H_DIM), lambda i: (i, 0)),
          pl.BlockSpec((NB, 3), lambda i: (i, 0)),
      ],
      out_shape=[
          jax.ShapeDtypeStruct((N_PAD, H_DIM), _f32),
          jax.ShapeDtypeStruct((N_PAD, 3), _f32),
      ],
      compiler_params=pltpu.CompilerParams(
          dimension_semantics=("arbitrary",)),
  )(h_pad, magg_a, magg_a, magg_b, magg_b, cagg_a, cagg_b, xp_pad,
    Wn1[:H_DIM], Wn1[H_DIM:], bn1.reshape(1, -1), Wn2, bn2.reshape(1, -1))

  return (h_out[:N_NODES], x_out[:N_NODES])


# speedup vs baseline: 1.4214x; 1.3467x over previous
import functools

import jax
import jax.numpy as jnp
from jax import lax
from jax.experimental import pallas as pl
from jax.experimental.pallas import tpu as pltpu
from jax.experimental.pallas import tpu_sc as plsc

N_NODES = 10000
N_PAD = 10240
E_EDGES = 320000
H_DIM = 128
NC, NS = 2, 16
NW = NC * NS
CHUNK = 128
ROWS_PER_SUB = N_PAD // NS
EB = 512
CPR = EB // CHUNK
E_A = 312 * EB
E_B = E_EDGES - E_A

_f32 = jnp.float32
_i32 = jnp.int32
_mesh = plsc.VectorSubcoreMesh(core_axis_name="c", subcore_axis_name="s")


def _make_gather(ne):
  nchunk = ne // CHUNK
  steps = (nchunk + NW - 1) // NW
  erows = ne // EB

  @functools.partial(
      pl.kernel,
      out_type=(
          jax.ShapeDtypeStruct((ne, H_DIM), _f32),
          jax.ShapeDtypeStruct((ne, H_DIM), _f32),
          jax.ShapeDtypeStruct((erows, 1, EB), _f32),
          jax.ShapeDtypeStruct((erows, 1, EB), _f32),
          jax.ShapeDtypeStruct((erows, 1, EB), _f32),
          jax.ShapeDtypeStruct((erows, 1, EB), _f32),
      ),
      mesh=_mesh,
      scratch_types=[
          pltpu.VMEM((4 * N_NODES,), _f32),
          pltpu.VMEM((CHUNK,), _i32),
          pltpu.VMEM((CHUNK,), _i32),
          pltpu.VMEM((CHUNK, H_DIM), _f32),
          pltpu.VMEM((CHUNK, H_DIM), _f32),
          pltpu.VMEM((1, CHUNK), _f32),
          pltpu.VMEM((1, CHUNK), _f32),
          pltpu.VMEM((1, CHUNK), _f32),
          pltpu.VMEM((1, CHUNK), _f32),
          pltpu.SemaphoreType.DMA,
      ],
      compiler_params=pltpu.CompilerParams(needs_layout_passes=False),
  )
  def gather_k(p_hbm, xflat_hbm, row_hbm, col_hbm, q_hbm,
               hi_out, hj_out, dx_out, dy_out, dz_out, d2_out,
               xtab, rowv, colv, hbi, hbj, dxb, dyb, dzb, d2b, sem):
    wid = lax.axis_index("s") * NC + lax.axis_index("c")
    pltpu.sync_copy(xflat_hbm, xtab)

    def body(t, _):
      ci = wid + t * NW

      @pl.when(ci < nchunk)
      def _():
        base = ci * CHUNK
        pltpu.sync_copy(row_hbm.at[pl.ds(base, CHUNK)], rowv)
        pltpu.sync_copy(col_hbm.at[pl.ds(base, CHUNK)], colv)
        c1 = pltpu.async_copy(p_hbm.at[colv], hbi, sem)
        c2 = pltpu.async_copy(q_hbm.at[rowv], hbj, sem)
        for g in range(CHUNK // 16):
          r16 = rowv[pl.ds(g * 16, 16)] * 4
          c16 = colv[pl.ds(g * 16, 16)] * 4
          dx = plsc.load_gather(xtab, [c16]) - plsc.load_gather(xtab, [r16])
          dy = (plsc.load_gather(xtab, [c16 + 1])
                - plsc.load_gather(xtab, [r16 + 1]))
          dz = (plsc.load_gather(xtab, [c16 + 2])
                - plsc.load_gather(xtab, [r16 + 2]))
          dxb[0, pl.ds(g * 16, 16)] = dx
          dyb[0, pl.ds(g * 16, 16)] = dy
          dzb[0, pl.ds(g * 16, 16)] = dz
          d2b[0, pl.ds(g * 16, 16)] = dx * dx + dy * dy + dz * dz
        c1.wait()
        c2.wait()
        pltpu.sync_copy(hbi, hi_out.at[pl.ds(base, CHUNK)])
        pltpu.sync_copy(hbj, hj_out.at[pl.ds(base, CHUNK)])
        er = ci // CPR
        ec = (ci % CPR) * CHUNK
        pltpu.sync_copy(dxb, dx_out.at[er, pl.ds(0, 1), pl.ds(ec, CHUNK)])
        pltpu.sync_copy(dyb, dy_out.at[er, pl.ds(0, 1), pl.ds(ec, CHUNK)])
        pltpu.sync_copy(dzb, dz_out.at[er, pl.ds(0, 1), pl.ds(ec, CHUNK)])
        pltpu.sync_copy(d2b, d2_out.at[er, pl.ds(0, 1), pl.ds(ec, CHUNK)])
      return 0

    lax.fori_loop(0, steps, body, 0)

  return gather_k


def _make_scatter(ne):
  nchunk = ne // CHUNK
  steps = (nchunk + NW - 1) // NW

  @functools.partial(
      pl.kernel,
      out_type=(
          jax.ShapeDtypeStruct((NC, N_PAD, H_DIM), _f32),
          jax.ShapeDtypeStruct((NW, 3 * N_PAD), _f32),
      ),
      mesh=_mesh,
      scratch_types=[
          pltpu.VMEM((CHUNK,), _i32),
          pltpu.VMEM((CHUNK, H_DIM), _f32),
          pltpu.VMEM((CHUNK,), _f32),
          pltpu.VMEM((CHUNK,), _f32),
          pltpu.VMEM((CHUNK,), _f32),
          pltpu.VMEM((CHUNK,), _f32),
          pltpu.VMEM((3 * N_PAD,), _f32),
          pltpu.VMEM_SHARED((N_PAD, H_DIM), _f32),
          pltpu.SemaphoreType.DMA,
      ],
      compiler_params=pltpu.CompilerParams(needs_layout_passes=False),
  )
  def scatter_k(m_hbm, cw_hbm, dx_hbm, dy_hbm, dz_hbm, col_hbm, z_hbm, zc_hbm,
                magg_out, cagg_out,
                colv, mbuf, cwb, dxb, dyb, dzb, cacc, macc, sem):
    cid = lax.axis_index("c")
    sid = lax.axis_index("s")
    wid = sid * NC + cid
    rbase = sid * ROWS_PER_SUB

    pltpu.sync_copy(z_hbm.at[pl.ds(rbase, ROWS_PER_SUB)],
                    macc.at[pl.ds(rbase, ROWS_PER_SUB)])
    pltpu.sync_copy(zc_hbm, cacc)
    plsc.subcore_barrier()

    def body(t, _):
      ci = wid + t * NW

      @pl.when(ci < nchunk)
      def _():
        base = ci * CHUNK
        pltpu.sync_copy(col_hbm.at[pl.ds(base, CHUNK)], colv)
        pltpu.sync_copy(m_hbm.at[pl.ds(base, CHUNK)], mbuf)
        er = ci // CPR
        ec = (ci % CPR) * CHUNK
        pltpu.sync_copy(cw_hbm.at[er, 0, pl.ds(ec, CHUNK)], cwb)
        pltpu.sync_copy(dx_hbm.at[er, 0, pl.ds(ec, CHUNK)], dxb)
        pltpu.sync_copy(dy_hbm.at[er, 0, pl.ds(ec, CHUNK)], dyb)
        pltpu.sync_copy(dz_hbm.at[er, 0, pl.ds(ec, CHUNK)], dzb)
        pltpu.sync_copy(mbuf, macc.at[colv], add=True)
        for g in range(CHUNK // 16):
          sl = pl.ds(g * 16, 16)
          c3 = colv[sl] * 3
          cw = cwb[sl]
          plsc.addupdate_scatter(cacc, [c3], dxb[sl] * cw)
          plsc.addupdate_scatter(cacc, [c3 + 1], dyb[sl] * cw)
          plsc.addupdate_scatter(cacc, [c3 + 2], dzb[sl] * cw)
      return 0

    lax.fori_loop(0, steps, body, 0)
    plsc.subcore_barrier()

    pltpu.sync_copy(macc.at[pl.ds(rbase, ROWS_PER_SUB)],
                    magg_out.at[cid, pl.ds(rbase, ROWS_PER_SUB)])
    pltpu.sync_copy(cacc, cagg_out.at[wid])

  return scatter_k


_GATHER = {E_A: _make_gather(E_A), E_B: _make_gather(E_B)}
_SCATTER = {E_A: _make_scatter(E_A), E_B: _make_scatter(E_B)}


def _prep_body(h, A, B, p_out, q_out):
  p_out[...] = jnp.dot(h[...], A[...], preferred_element_type=_f32)
  q_out[...] = jnp.dot(h[...], B[...], preferred_element_type=_f32)


def _edge_body(hi, hj, d2, ea, mask,
               wd, C, be1, We2, be2, wg, bg, wc, bc,
               m_out, cw_out):
  d2c = d2[...].reshape(EB, 1)
  dist = jnp.sqrt(d2c + 1e-8)
  t = (hi[...] + hj[...]
       + dist * wd[...]
       + jnp.dot(ea[...], C[...], preferred_element_type=_f32)
       + be1[...])
  m1 = t * jax.nn.sigmoid(t)
  u = jnp.dot(m1, We2[...], preferred_element_type=_f32) + be2[...]
  m2 = u * jax.nn.sigmoid(u)
  gate = jax.nn.sigmoid(
      jnp.sum(m2 * wg[...], axis=-1, keepdims=True) + bg[...])
  mg = m2 * (gate * mask[...].reshape(EB, 1))
  cw = jnp.sum(mg * wc[...], axis=-1, keepdims=True) + bc[...]
  m_out[...] = mg
  cw_out[...] = cw.reshape(1, 1, EB)


def _full_spec(shape):
  return pl.BlockSpec(shape, lambda i: tuple(0 for _ in shape))


def _edge_call(hi, hj, d2, ea, mask, weights):
  ne = hi.shape[0]
  erows = ne // EB
  return pl.pallas_call(
      _edge_body,
      grid=(erows,),
      in_specs=[
          pl.BlockSpec((EB, H_DIM), lambda i: (i, 0)),
          pl.BlockSpec((EB, H_DIM), lambda i: (i, 0)),
          pl.BlockSpec((1, 1, EB), lambda i: (i, 0, 0)),
          pl.BlockSpec((EB, 16), lambda i: (i, 0)),
          pl.BlockSpec((1, 1, EB), lambda i: (i, 0, 0)),
          _full_spec((1, H_DIM)),
          _full_spec((16, H_DIM)),
          _full_spec((1, H_DIM)),
          _full_spec((H_DIM, H_DIM)),
          _full_spec((1, H_DIM)),
          _full_spec((1, H_DIM)),
          _full_spec((1, 1)),
          _full_spec((1, H_DIM)),
          _full_spec((1, 1)),
      ],
      out_specs=[
          pl.BlockSpec((EB, H_DIM), lambda i: (i, 0)),
          pl.BlockSpec((1, 1, EB), lambda i: (i, 0, 0)),
      ],
      out_shape=[
          jax.ShapeDtypeStruct((ne, H_DIM), _f32),
          jax.ShapeDtypeStruct((erows, 1, EB), _f32),
      ],
      compiler_params=pltpu.CompilerParams(
          dimension_semantics=("arbitrary",)),
  )(hi, hj, d2, ea, mask, *weights)


def _node_body(h, pa0, pa1, pb0, pb1, cpa, cpb, xp,
               Wn1a, Wn1b, bn1, Wn2, bn2,
               h_out, x_out):
  ma = (pa0[0] + pa1[0]) + (pb0[0] + pb1[0])
  t = (jnp.dot(h[...], Wn1a[...], preferred_element_type=_f32)
       + jnp.dot(ma, Wn1b[...], preferred_element_type=_f32)
       + bn1[...])
  s = t * jax.nn.sigmoid(t)
  dh = jnp.dot(s, Wn2[...], preferred_element_type=_f32) + bn2[...]
  h_out[...] = h[...] + dh
  cx = jnp.sum(cpa[...], axis=0) + jnp.sum(cpb[...], axis=0)
  x_out[...] = xp[...] + cx.reshape(1, 1, -1)


def kernel(h, x, edge_index, edge_mask, edge_attr,
           We1, be1, We2, be2, Wg, bg, Wn1, bn1, Wn2, bn2, Wc, bc):
  row = edge_index[0]
  col = edge_index[1]
  x_flat = jnp.pad(x, ((0, 0), (0, 1))).reshape(-1)

  A = We1[:H_DIM]
  B = We1[H_DIM:2 * H_DIM]
  wd = We1[2 * H_DIM:2 * H_DIM + 1]
  C = We1[2 * H_DIM + 1:]
  ew = (wd, C, be1.reshape(1, -1), We2, be2.reshape(1, -1),
        Wg.reshape(1, -1), bg.reshape(1, 1), Wc.reshape(1, -1),
        bc.reshape(1, 1))

  h_pad = jnp.pad(h, ((0, N_PAD - N_NODES), (0, 0)))
  NB = 512
  ngrid = N_PAD // NB
  P, Q = pl.pallas_call(
      _prep_body,
      grid=(ngrid,),
      in_specs=[
          pl.BlockSpec((NB, H_DIM), lambda i: (i, 0)),
          _full_spec((H_DIM, H_DIM)),
          _full_spec((H_DIM, H_DIM)),
      ],
      out_specs=[
          pl.BlockSpec((NB, H_DIM), lambda i: (i, 0)),
          pl.BlockSpec((NB, H_DIM), lambda i: (i, 0)),
      ],
      out_shape=[
          jax.ShapeDtypeStruct((N_PAD, H_DIM), _f32),
          jax.ShapeDtypeStruct((N_PAD, H_DIM), _f32),
      ],
      compiler_params=pltpu.CompilerParams(
          dimension_semantics=("arbitrary",)),
  )(h_pad, A, B)

  zeros = jnp.zeros((N_PAD, H_DIM), _f32)
  zeros_c = jnp.zeros((3 * N_PAD,), _f32)

  rows = (row[:E_A], row[E_A:])
  cols = (col[:E_A], col[E_A:])
  eas = (edge_attr[:E_A], edge_attr[E_A:])
  mask_r = edge_mask.reshape(E_EDGES // EB, 1, EB)
  masks = (mask_r[:E_A // EB], mask_r[E_A // EB:])
  sizes = (E_A, E_B)

  gat = [None, None]
  emlp = [None, None]
  scat = [None, None]
  gat[0] = _GATHER[E_A](P, x_flat, rows[0], cols[0], Q)
  gat[1] = _GATHER[E_B](P, x_flat, rows[1], cols[1], Q)
  for half in range(2):
    hi, hj, dx, dy, dz, d2 = gat[half]
    emlp[half] = _edge_call(hi, hj, d2, eas[half], masks[half], ew)
    m_ij, cw = emlp[half]
    scat[half] = _SCATTER[sizes[half]](m_ij, cw, dx, dy, dz, cols[half],
                                       zeros, zeros_c)

  magg_a, cagg_a = scat[0]
  magg_b, cagg_b = scat[1]

  xp_flat = jnp.pad(x, ((0, N_PAD - N_NODES), (0, 0))).reshape(ngrid, 1, -1)

  h_out, x_out = pl.pallas_call(
      _node_body,
      grid=(ngrid,),
      in_specs=[
          pl.BlockSpec((NB, H_DIM), lambda i: (i, 0)),
          pl.BlockSpec((1, NB, H_DIM), lambda i: (0, i, 0)),
          pl.BlockSpec((1, NB, H_DIM), lambda i: (1, i, 0)),
          pl.BlockSpec((1, NB, H_DIM), lambda i: (0, i, 0)),
          pl.BlockSpec((1, NB, H_DIM), lambda i: (1, i, 0)),
          pl.BlockSpec((NW, 3 * NB), lambda i: (0, i)),
          pl.BlockSpec((NW, 3 * NB), lambda i: (0, i)),
          pl.BlockSpec((1, 1, 3 * NB), lambda i: (i, 0, 0)),
          _full_spec((H_DIM, H_DIM)),
          _full_spec((H_DIM, H_DIM)),
          _full_spec((1, H_DIM)),
          _full_spec((H_DIM, H_DIM)),
          _full_spec((1, H_DIM)),
      ],
      out_specs=[
          pl.BlockSpec((NB, H_DIM), lambda i: (i, 0)),
          pl.BlockSpec((1, 1, 3 * NB), lambda i: (i, 0, 0)),
      ],
      out_shape=[
          jax.ShapeDtypeStruct((N_PAD, H_DIM), _f32),
          jax.ShapeDtypeStruct((ngrid, 1, 3 * NB), _f32),
      ],
      compiler_params=pltpu.CompilerParams(
          dimension_semantics=("arbitrary",)),
  )(h_pad, magg_a, magg_a, magg_b, magg_b, cagg_a, cagg_b, xp_flat,
    Wn1[:H_DIM], Wn1[H_DIM:], bn1.reshape(1, -1), Wn2, bn2.reshape(1, -1))

  return (h_out[:N_NODES], x_out.reshape(N_PAD, 3)[:N_NODES])


# speedup vs baseline: 1.5584x; 1.0964x over previous
import functools

import jax
import jax.numpy as jnp
from jax import lax
from jax.experimental import pallas as pl
from jax.experimental.pallas import tpu as pltpu
from jax.experimental.pallas import tpu_sc as plsc

N_NODES = 10000
N_PAD = 10240
E_EDGES = 320000
H_DIM = 128
NC, NS = 2, 16
NW = NC * NS
CHUNK = 128
ROWS_PER_SUB = N_PAD // NS
EB = 512
CPR = EB // CHUNK
E_A = 312 * EB
E_B = E_EDGES - E_A

_f32 = jnp.float32
_i32 = jnp.int32
_mesh = plsc.VectorSubcoreMesh(core_axis_name="c", subcore_axis_name="s")


def _make_gather(ne):
  nchunk = ne // CHUNK
  steps = (nchunk + NW - 1) // NW
  erows = ne // EB

  @functools.partial(
      pl.kernel,
      out_type=(
          jax.ShapeDtypeStruct((ne, H_DIM), _f32),
          jax.ShapeDtypeStruct((ne, H_DIM), _f32),
          jax.ShapeDtypeStruct((erows, 1, EB), _f32),
          jax.ShapeDtypeStruct((erows, 1, EB), _f32),
          jax.ShapeDtypeStruct((erows, 1, EB), _f32),
          jax.ShapeDtypeStruct((erows, 1, EB), _f32),
      ),
      mesh=_mesh,
      scratch_types=[
          pltpu.VMEM((4 * N_NODES,), _f32),
          pltpu.VMEM((CHUNK,), _i32),
          pltpu.VMEM((CHUNK,), _i32),
          pltpu.VMEM((CHUNK, H_DIM), _f32),
          pltpu.VMEM((CHUNK, H_DIM), _f32),
          pltpu.VMEM((1, CHUNK), _f32),
          pltpu.VMEM((1, CHUNK), _f32),
          pltpu.VMEM((1, CHUNK), _f32),
          pltpu.VMEM((1, CHUNK), _f32),
          pltpu.SemaphoreType.DMA,
      ],
      compiler_params=pltpu.CompilerParams(needs_layout_passes=False),
  )
  def gather_k(p_hbm, xflat_hbm, row_hbm, col_hbm, q_hbm,
               hi_out, hj_out, dx_out, dy_out, dz_out, d2_out,
               xtab, rowv, colv, hbi, hbj, dxb, dyb, dzb, d2b, sem):
    wid = lax.axis_index("s") * NC + lax.axis_index("c")
    pltpu.sync_copy(xflat_hbm, xtab)

    def body(t, _):
      ci = wid + t * NW

      @pl.when(ci < nchunk)
      def _():
        base = ci * CHUNK
        i1 = pltpu.async_copy(row_hbm.at[pl.ds(base, CHUNK)], rowv, sem)
        i2 = pltpu.async_copy(col_hbm.at[pl.ds(base, CHUNK)], colv, sem)
        i1.wait()
        i2.wait()
        c1 = pltpu.async_copy(p_hbm.at[colv], hbi, sem)
        c2 = pltpu.async_copy(q_hbm.at[rowv], hbj, sem)
        for g in range(CHUNK // 16):
          r16 = rowv[pl.ds(g * 16, 16)] * 4
          c16 = colv[pl.ds(g * 16, 16)] * 4
          dx = plsc.load_gather(xtab, [c16]) - plsc.load_gather(xtab, [r16])
          dy = (plsc.load_gather(xtab, [c16 + 1])
                - plsc.load_gather(xtab, [r16 + 1]))
          dz = (plsc.load_gather(xtab, [c16 + 2])
                - plsc.load_gather(xtab, [r16 + 2]))
          dxb[0, pl.ds(g * 16, 16)] = dx
          dyb[0, pl.ds(g * 16, 16)] = dy
          dzb[0, pl.ds(g * 16, 16)] = dz
          d2b[0, pl.ds(g * 16, 16)] = dx * dx + dy * dy + dz * dz
        er = ci // CPR
        ec = (ci % CPR) * CHUNK
        w3 = pltpu.async_copy(dxb, dx_out.at[er, pl.ds(0, 1),
                                             pl.ds(ec, CHUNK)], sem)
        w4 = pltpu.async_copy(dyb, dy_out.at[er, pl.ds(0, 1),
                                             pl.ds(ec, CHUNK)], sem)
        w5 = pltpu.async_copy(dzb, dz_out.at[er, pl.ds(0, 1),
                                             pl.ds(ec, CHUNK)], sem)
        w6 = pltpu.async_copy(d2b, d2_out.at[er, pl.ds(0, 1),
                                             pl.ds(ec, CHUNK)], sem)
        c1.wait()
        c2.wait()
        w1 = pltpu.async_copy(hbi, hi_out.at[pl.ds(base, CHUNK)], sem)
        w2 = pltpu.async_copy(hbj, hj_out.at[pl.ds(base, CHUNK)], sem)
        w3.wait()
        w4.wait()
        w5.wait()
        w6.wait()
        w1.wait()
        w2.wait()
      return 0

    lax.fori_loop(0, steps, body, 0)

  return gather_k


def _make_scatter(ne):
  nchunk = ne // CHUNK
  steps = (nchunk + NW - 1) // NW

  @functools.partial(
      pl.kernel,
      out_type=(
          jax.ShapeDtypeStruct((NC, N_PAD, H_DIM), _f32),
          jax.ShapeDtypeStruct((NW, 3 * N_PAD), _f32),
      ),
      mesh=_mesh,
      scratch_types=[
          pltpu.VMEM((CHUNK,), _i32),
          pltpu.VMEM((CHUNK, H_DIM), _f32),
          pltpu.VMEM((CHUNK,), _f32),
          pltpu.VMEM((CHUNK,), _f32),
          pltpu.VMEM((CHUNK,), _f32),
          pltpu.VMEM((CHUNK,), _f32),
          pltpu.VMEM((3 * N_PAD,), _f32),
          pltpu.VMEM_SHARED((N_PAD, H_DIM), _f32),
          pltpu.SemaphoreType.DMA,
      ],
      compiler_params=pltpu.CompilerParams(needs_layout_passes=False),
  )
  def scatter_k(m_hbm, cw_hbm, dx_hbm, dy_hbm, dz_hbm, col_hbm, z_hbm, zc_hbm,
                magg_out, cagg_out,
                colv, mbuf, cwb, dxb, dyb, dzb, cacc, macc, sem):
    cid = lax.axis_index("c")
    sid = lax.axis_index("s")
    wid = sid * NC + cid
    rbase = sid * ROWS_PER_SUB

    pltpu.sync_copy(z_hbm.at[pl.ds(rbase, ROWS_PER_SUB)],
                    macc.at[pl.ds(rbase, ROWS_PER_SUB)])
    pltpu.sync_copy(zc_hbm, cacc)
    plsc.subcore_barrier()

    def body(t, _):
      ci = wid + t * NW

      @pl.when(ci < nchunk)
      def _():
        base = ci * CHUNK
        er = ci // CPR
        ec = (ci % CPR) * CHUNK
        i1 = pltpu.async_copy(col_hbm.at[pl.ds(base, CHUNK)], colv, sem)
        i2 = pltpu.async_copy(m_hbm.at[pl.ds(base, CHUNK)], mbuf, sem)
        i3 = pltpu.async_copy(cw_hbm.at[er, 0, pl.ds(ec, CHUNK)], cwb, sem)
        i4 = pltpu.async_copy(dx_hbm.at[er, 0, pl.ds(ec, CHUNK)], dxb, sem)
        i5 = pltpu.async_copy(dy_hbm.at[er, 0, pl.ds(ec, CHUNK)], dyb, sem)
        i6 = pltpu.async_copy(dz_hbm.at[er, 0, pl.ds(ec, CHUNK)], dzb, sem)
        i1.wait()
        i2.wait()
        i3.wait()
        i4.wait()
        i5.wait()
        i6.wait()
        pltpu.sync_copy(mbuf, macc.at[colv], add=True)
        for g in range(CHUNK // 16):
          sl = pl.ds(g * 16, 16)
          c3 = colv[sl] * 3
          cw = cwb[sl]
          plsc.addupdate_scatter(cacc, [c3], dxb[sl] * cw)
          plsc.addupdate_scatter(cacc, [c3 + 1], dyb[sl] * cw)
          plsc.addupdate_scatter(cacc, [c3 + 2], dzb[sl] * cw)
      return 0

    lax.fori_loop(0, steps, body, 0)
    plsc.subcore_barrier()

    pltpu.sync_copy(macc.at[pl.ds(rbase, ROWS_PER_SUB)],
                    magg_out.at[cid, pl.ds(rbase, ROWS_PER_SUB)])
    pltpu.sync_copy(cacc, cagg_out.at[wid])

  return scatter_k


_GATHER = {E_A: _make_gather(E_A), E_B: _make_gather(E_B)}
_SCATTER = {E_A: _make_scatter(E_A), E_B: _make_scatter(E_B)}


def _prep_body(h, A, B, p_out, q_out):
  p_out[...] = jnp.dot(h[...], A[...], preferred_element_type=_f32)
  q_out[...] = jnp.dot(h[...], B[...], preferred_element_type=_f32)


def _edge_body(hi, hj, d2, ea, mask,
               wd, C, be1, We2, be2, wg, bg, wc, bc,
               m_out, cw_out):
  d2c = d2[...].reshape(EB, 1)
  dist = jnp.sqrt(d2c + 1e-8)
  t = (hi[...] + hj[...]
       + dist * wd[...]
       + jnp.dot(ea[...], C[...], preferred_element_type=_f32)
       + be1[...])
  m1 = t * jax.nn.sigmoid(t)
  u = jnp.dot(m1, We2[...], preferred_element_type=_f32) + be2[...]
  m2 = u * jax.nn.sigmoid(u)
  gate = jax.nn.sigmoid(
      jnp.sum(m2 * wg[...], axis=-1, keepdims=True) + bg[...])
  mg = m2 * (gate * mask[...].reshape(EB, 1))
  cw = jnp.sum(mg * wc[...], axis=-1, keepdims=True) + bc[...]
  m_out[...] = mg
  cw_out[...] = cw.reshape(1, 1, EB)


def _full_spec(shape):
  return pl.BlockSpec(shape, lambda i: tuple(0 for _ in shape))


def _edge_call(hi, hj, d2, ea, mask, weights):
  ne = hi.shape[0]
  erows = ne // EB
  return pl.pallas_call(
      _edge_body,
      grid=(erows,),
      in_specs=[
          pl.BlockSpec((EB, H_DIM), lambda i: (i, 0)),
          pl.BlockSpec((EB, H_DIM), lambda i: (i, 0)),
          pl.BlockSpec((1, 1, EB), lambda i: (i, 0, 0)),
          pl.BlockSpec((EB, 16), lambda i: (i, 0)),
          pl.BlockSpec((1, 1, EB), lambda i: (i, 0, 0)),
          _full_spec((1, H_DIM)),
          _full_spec((16, H_DIM)),
          _full_spec((1, H_DIM)),
          _full_spec((H_DIM, H_DIM)),
          _full_spec((1, H_DIM)),
          _full_spec((1, H_DIM)),
          _full_spec((1, 1)),
          _full_spec((1, H_DIM)),
          _full_spec((1, 1)),
      ],
      out_specs=[
          pl.BlockSpec((EB, H_DIM), lambda i: (i, 0)),
          pl.BlockSpec((1, 1, EB), lambda i: (i, 0, 0)),
      ],
      out_shape=[
          jax.ShapeDtypeStruct((ne, H_DIM), _f32),
          jax.ShapeDtypeStruct((erows, 1, EB), _f32),
      ],
      compiler_params=pltpu.CompilerParams(
          dimension_semantics=("arbitrary",)),
  )(hi, hj, d2, ea, mask, *weights)


def _node_body(h, pa0, pa1, pb0, pb1, cpa, cpb, xp,
               Wn1a, Wn1b, bn1, Wn2, bn2,
               h_out, x_out):
  ma = (pa0[0] + pa1[0]) + (pb0[0] + pb1[0])
  t = (jnp.dot(h[...], Wn1a[...], preferred_element_type=_f32)
       + jnp.dot(ma, Wn1b[...], preferred_element_type=_f32)
       + bn1[...])
  s = t * jax.nn.sigmoid(t)
  dh = jnp.dot(s, Wn2[...], preferred_element_type=_f32) + bn2[...]
  h_out[...] = h[...] + dh
  cx = jnp.sum(cpa[...], axis=0) + jnp.sum(cpb[...], axis=0)
  x_out[...] = xp[...] + cx.reshape(1, 1, -1)


def kernel(h, x, edge_index, edge_mask, edge_attr,
           We1, be1, We2, be2, Wg, bg, Wn1, bn1, Wn2, bn2, Wc, bc):
  row = edge_index[0]
  col = edge_index[1]
  x_flat = jnp.pad(x, ((0, 0), (0, 1))).reshape(-1)

  A = We1[:H_DIM]
  B = We1[H_DIM:2 * H_DIM]
  wd = We1[2 * H_DIM:2 * H_DIM + 1]
  C = We1[2 * H_DIM + 1:]
  ew = (wd, C, be1.reshape(1, -1), We2, be2.reshape(1, -1),
        Wg.reshape(1, -1), bg.reshape(1, 1), Wc.reshape(1, -1),
        bc.reshape(1, 1))

  h_pad = jnp.pad(h, ((0, N_PAD - N_NODES), (0, 0)))
  NB = 512
  ngrid = N_PAD // NB
  P, Q = pl.pallas_call(
      _prep_body,
      grid=(ngrid,),
      in_specs=[
          pl.BlockSpec((NB, H_DIM), lambda i: (i, 0)),
          _full_spec((H_DIM, H_DIM)),
          _full_spec((H_DIM, H_DIM)),
      ],
      out_specs=[
          pl.BlockSpec((NB, H_DIM), lambda i: (i, 0)),
          pl.BlockSpec((NB, H_DIM), lambda i: (i, 0)),
      ],
      out_shape=[
          jax.ShapeDtypeStruct((N_PAD, H_DIM), _f32),
          jax.ShapeDtypeStruct((N_PAD, H_DIM), _f32),
      ],
      compiler_params=pltpu.CompilerParams(
          dimension_semantics=("arbitrary",)),
  )(h_pad, A, B)

  zeros = jnp.zeros((N_PAD, H_DIM), _f32)
  zeros_c = jnp.zeros((3 * N_PAD,), _f32)

  rows = (row[:E_A], row[E_A:])
  cols = (col[:E_A], col[E_A:])
  eas = (edge_attr[:E_A], edge_attr[E_A:])
  mask_r = edge_mask.reshape(E_EDGES // EB, 1, EB)
  masks = (mask_r[:E_A // EB], mask_r[E_A // EB:])
  sizes = (E_A, E_B)

  gat = [None, None]
  emlp = [None, None]
  scat = [None, None]
  gat[0] = _GATHER[E_A](P, x_flat, rows[0], cols[0], Q)
  gat[1] = _GATHER[E_B](P, x_flat, rows[1], cols[1], Q)
  for half in range(2):
    hi, hj, dx, dy, dz, d2 = gat[half]
    emlp[half] = _edge_call(hi, hj, d2, eas[half], masks[half], ew)
    m_ij, cw = emlp[half]
    scat[half] = _SCATTER[sizes[half]](m_ij, cw, dx, dy, dz, cols[half],
                                       zeros, zeros_c)

  magg_a, cagg_a = scat[0]
  magg_b, cagg_b = scat[1]

  xp_flat = jnp.pad(x, ((0, N_PAD - N_NODES), (0, 0))).reshape(ngrid, 1, -1)

  h_out, x_out = pl.pallas_call(
      _node_body,
      grid=(ngrid,),
      in_specs=[
          pl.BlockSpec((NB, H_DIM), lambda i: (i, 0)),
          pl.BlockSpec((1, NB, H_DIM), lambda i: (0, i, 0)),
          pl.BlockSpec((1, NB, H_DIM), lambda i: (1, i, 0)),
          pl.BlockSpec((1, NB, H_DIM), lambda i: (0, i, 0)),
          pl.BlockSpec((1, NB, H_DIM), lambda i: (1, i, 0)),
          pl.BlockSpec((NW, 3 * NB), lambda i: (0, i)),
          pl.BlockSpec((NW, 3 * NB), lambda i: (0, i)),
          pl.BlockSpec((1, 1, 3 * NB), lambda i: (i, 0, 0)),
          _full_spec((H_DIM, H_DIM)),
          _full_spec((H_DIM, H_DIM)),
          _full_spec((1, H_DIM)),
          _full_spec((H_DIM, H_DIM)),
          _full_spec((1, H_DIM)),
      ],
      out_specs=[
          pl.BlockSpec((NB, H_DIM), lambda i: (i, 0)),
          pl.BlockSpec((1, 1, 3 * NB), lambda i: (i, 0, 0)),
      ],
      out_shape=[
          jax.ShapeDtypeStruct((N_PAD, H_DIM), _f32),
          jax.ShapeDtypeStruct((ngrid, 1, 3 * NB), _f32),
      ],
      compiler_params=pltpu.CompilerParams(
          dimension_semantics=("arbitrary",)),
  )(h_pad, magg_a, magg_a, magg_b, magg_b, cagg_a, cagg_b, xp_flat,
    Wn1[:H_DIM], Wn1[H_DIM:], bn1.reshape(1, -1), Wn2, bn2.reshape(1, -1))

  return (h_out[:N_NODES], x_out.reshape(N_PAD, 3)[:N_NODES])
